# Initial kernel scaffold; baseline (speedup 1.0000x reference)
import jax
import jax.numpy as jnp
from jax import lax
from jax.experimental import pallas as pl
from jax.experimental.pallas import tpu as pltpu
from jax.experimental.pallas import tpu_sc as plsc

_N = 10000
_E = 320000
_D = 128
_H = 16
_GAMMA = -0.5
_ZETA = 1.1
_NC = 2
_NS = 16
_NW = _NC * _NS
_EPW = _E // _NW
_L = 16


def _prep_body(x_ref, nbW_ref, nbb_ref, sW_ref, sb_ref, aw1_ref, aw2_ref,
               ab_ref, noise_ref, e1_ref, e2_ref, c_ref):
    x = x_ref[...]
    ab = ab_ref[0, 0]
    i1 = jnp.maximum(x @ nbW_ref[...] + nbb_ref[...], 0.0)
    i2 = jnp.maximum(x @ sW_ref[...] + sb_ref[...], 0.0)
    a1 = i1 @ aw1_ref[...]
    a2 = i2 @ aw2_ref[...]
    e1_ref[...] = jnp.exp(-(a1 + 0.5 * ab))
    e2_ref[...] = jnp.exp(-(a2 + 0.5 * ab))
    rn = noise_ref[...] + 1e-7
    c_ref[...] = (1.0 - rn) / rn


def _edge1_body(e1_hbm, e2_hbm, rows_hbm, cols_hbm, c_hbm, adj_hbm,
                masked_hbm, rowsum_hbm,
                e1_v, e2_v, rows_v, cols_v, c_v, adj_v, masked_v, rowsum_v):
    cid = lax.axis_index("c")
    sid = lax.axis_index("s")
    wid = sid * _NC + cid
    base = wid * _EPW
    pltpu.sync_copy(e1_hbm, e1_v)
    pltpu.sync_copy(e2_hbm, e2_v)
    pltpu.sync_copy(rows_hbm.at[pl.ds(base, _EPW)], rows_v)
    pltpu.sync_copy(cols_hbm.at[pl.ds(base, _EPW)], cols_v)
    pltpu.sync_copy(c_hbm.at[pl.ds(base, _EPW)], c_v)
    pltpu.sync_copy(adj_hbm.at[pl.ds(base, _EPW)], adj_v)

    def zero_body(i, carry):
        rowsum_v[pl.ds(i * _L, _L)] = jnp.zeros((_L,), jnp.float32)
        return carry
    lax.fori_loop(0, _N // _L, zero_body, 0)

    def body(i, carry):
        s = pl.ds(i * _L, _L)
        r16 = rows_v[s]
        k16 = cols_v[s]
        e1r = plsc.load_gather(e1_v, [r16])
        e2c = plsc.load_gather(e2_v, [k16])
        t = c_v[s] * e1r * e2c
        gate = 1.0 / (1.0 + t)
        m = jnp.clip(gate * (_ZETA - _GAMMA) + _GAMMA, 0.0, 1.0)
        mv = adj_v[s] * m
        masked_v[s] = mv
        plsc.addupdate_scatter(rowsum_v, [r16], mv)
        return carry
    lax.fori_loop(0, _EPW // _L, body, 0)

    pltpu.sync_copy(masked_v, masked_hbm.at[pl.ds(base, _EPW)])
    pltpu.sync_copy(rowsum_v, rowsum_hbm.at[wid])


def _norm_body(rowsum_ref, dinv_ref):
    s = jnp.sum(rowsum_ref[...], axis=0, keepdims=True) + 1e-10
    dinv_ref[...] = lax.rsqrt(s)


def _edge2_body(dinv_hbm, rows_hbm, cols_hbm, masked_hbm, out_hbm,
                dinv_v, rows_v, cols_v, masked_v, out_v):
    cid = lax.axis_index("c")
    sid = lax.axis_index("s")
    wid = sid * _NC + cid
    base = wid * _EPW
    pltpu.sync_copy(dinv_hbm, dinv_v)
    pltpu.sync_copy(rows_hbm.at[pl.ds(base, _EPW)], rows_v)
    pltpu.sync_copy(cols_hbm.at[pl.ds(base, _EPW)], cols_v)
    pltpu.sync_copy(masked_hbm.at[pl.ds(base, _EPW)], masked_v)

    def body(i, carry):
        s = pl.ds(i * _L, _L)
        dr = plsc.load_gather(dinv_v, [rows_v[s]])
        dc = plsc.load_gather(dinv_v, [cols_v[s]])
        out_v[s] = masked_v[s] * dr * dc
        return carry
    lax.fori_loop(0, _EPW // _L, body, 0)

    pltpu.sync_copy(out_v, out_hbm.at[pl.ds(base, _EPW)])


_mesh = plsc.VectorSubcoreMesh(core_axis_name="c", subcore_axis_name="s",
                               num_cores=_NC, num_subcores=_NS)

_edge1 = pl.kernel(
    _edge1_body,
    out_type=[jax.ShapeDtypeStruct((_E,), jnp.float32),
              jax.ShapeDtypeStruct((_NW, _N), jnp.float32)],
    mesh=_mesh,
    scratch_types=[pltpu.VMEM((_N,), jnp.float32),
                   pltpu.VMEM((_N,), jnp.float32),
                   pltpu.VMEM((_EPW,), jnp.int32),
                   pltpu.VMEM((_EPW,), jnp.int32),
                   pltpu.VMEM((_EPW,), jnp.float32),
                   pltpu.VMEM((_EPW,), jnp.float32),
                   pltpu.VMEM((_EPW,), jnp.float32),
                   pltpu.VMEM((_N,), jnp.float32)],
)

_edge2 = pl.kernel(
    _edge2_body,
    out_type=jax.ShapeDtypeStruct((_E,), jnp.float32),
    mesh=_mesh,
    scratch_types=[pltpu.VMEM((_N,), jnp.float32),
                   pltpu.VMEM((_EPW,), jnp.int32),
                   pltpu.VMEM((_EPW,), jnp.int32),
                   pltpu.VMEM((_EPW,), jnp.float32),
                   pltpu.VMEM((_EPW,), jnp.float32)],
)


def kernel(x, edge_index, adj_values, noise, nb_W, nb_b, self_W, self_b,
           att_W, att_b):
    rows = edge_index[0]
    cols = edge_index[1]
    noise2d = noise.reshape(_E // _D, _D)

    e1_2d, e2_2d, c2d = pl.pallas_call(
        _prep_body,
        out_shape=[jax.ShapeDtypeStruct((_N, 1), jnp.float32),
                   jax.ShapeDtypeStruct((_N, 1), jnp.float32),
                   jax.ShapeDtypeStruct((_E // _D, _D), jnp.float32)],
    )(x, nb_W, nb_b.reshape(1, _H), self_W, self_b.reshape(1, _H),
      att_W[:_H], att_W[_H:], att_b.reshape(1, 1), noise2d)

    e1 = e1_2d.reshape(_N)
    e2 = e2_2d.reshape(_N)
    cvals = c2d.reshape(_E)

    masked, rowsum_p = _edge1(e1, e2, rows, cols, cvals, adj_values)

    dinv2d = pl.pallas_call(
        _norm_body,
        out_shape=jax.ShapeDtypeStruct((1, _N), jnp.float32),
    )(rowsum_p)
    dinv = dinv2d.reshape(_N)

    return _edge2(dinv, rows, cols, masked)


# speedup vs baseline: 146.2519x; 146.2519x over previous
import jax
import jax.numpy as jnp
from jax import lax
from jax.experimental import pallas as pl
from jax.experimental.pallas import tpu as pltpu
from jax.experimental.pallas import tpu_sc as plsc

_N = 10000
_NP = 10240
_E = 320000
_D = 128
_H = 16
_GAMMA = -0.5
_ZETA = 1.1
_NC = 2
_NS = 16
_NW = _NC * _NS
_L = 16
_NCH = _E // _D
_WIN = 79
_WE = _WIN * _D


def _prep_body(x_ref, W2_ref, b2_ref, aw_ref, ab_ref, e1_ref, e2_ref):
    ab = ab_ref[0, 0]
    xb = x_ref[...]
    iT = jnp.maximum(
        lax.dot_general(W2_ref[...], xb, (((0,), (1,)), ((), ())),
                        preferred_element_type=jnp.float32) + b2_ref[...],
        0.0)
    aw = aw_ref[...]
    a1T = lax.dot_general(aw[:_H], iT[:_H], (((0,), (0,)), ((), ())),
                          preferred_element_type=jnp.float32)
    a2T = lax.dot_general(aw[_H:], iT[_H:], (((0,), (0,)), ((), ())),
                          preferred_element_type=jnp.float32)
    e1_ref[...] = jnp.exp(-(a1T + 0.5 * ab)).reshape(40, _D)
    e2_ref[...] = jnp.exp(-(a2T + 0.5 * ab)).reshape(40, _D)


def _edge1_body(e1_hbm, e2_hbm, ei_hbm, noise_hbm, adj_hbm,
                masked_hbm, rowsum_hbm,
                e1_v, e2_v, ei_v, nz_v, adj_v, masked_v, rowsum_v, sem):
    cid = lax.axis_index("c")
    sid = lax.axis_index("s")
    wid = sid * _NC + cid
    own_lo = (wid * _NCH) // _NW * _D
    own_hi = ((wid + 1) * _NCH) // _NW * _D
    rb = jnp.minimum(own_lo // _D, _NCH - _WIN)
    ebase = pl.multiple_of(rb * _D, _D)
    cps = [pltpu.async_copy(e1_hbm, e1_v, sem),
           pltpu.async_copy(e2_hbm, e2_v, sem),
           pltpu.async_copy(ei_hbm.at[pl.ds(rb, _WIN)], ei_v, sem),
           pltpu.async_copy(noise_hbm.at[pl.ds(rb, _WIN)], nz_v, sem),
           pltpu.async_copy(adj_hbm.at[pl.ds(ebase, _WE)], adj_v, sem)]

    @plsc.parallel_loop(0, _NP // _L, unroll=8)
    def zero_body(i):
        rowsum_v[i >> 3, pl.ds((i & 7) * _L, _L)] = jnp.zeros((_L,),
                                                              jnp.float32)

    for cp in cps:
        cp.wait()

    lane = lax.iota(jnp.int32, _L)

    @plsc.parallel_loop(0, _WE // _L, unroll=8)
    def body(i):
        s = pl.ds(i * _L, _L)
        c8 = i >> 3
        sl = pl.ds((i & 7) * _L, _L)
        r16 = ei_v[c8, 0, sl]
        k16 = ei_v[c8, 1, sl]
        e1r = plsc.load_gather(e1_v, [r16])
        e2c = plsc.load_gather(e2_v, [k16])
        rn = nz_v[c8, 0, sl] + 1e-7
        q = e1r * e2c
        gate = rn / (rn + (1.0 - rn) * q)
        m = jnp.clip(gate * (_ZETA - _GAMMA) + _GAMMA, 0.0, 1.0)
        mv = adj_v[s] * m
        masked_v[s] = mv
        g = (ebase + i * _L) + lane
        own = jnp.logical_and(g >= own_lo, g < own_hi)
        plsc.addupdate_scatter(
            rowsum_v,
            [lax.shift_right_logical(r16, 7), jnp.bitwise_and(r16, 127)],
            mv, mask=own)

    ocp1 = pltpu.async_copy(masked_v, masked_hbm.at[pl.ds(ebase, _WE)], sem)
    ocp2 = pltpu.async_copy(rowsum_v, rowsum_hbm.at[wid], sem)
    ocp1.wait()
    ocp2.wait()


def _norm_body(rowsum_ref, dinv_ref):
    s = jnp.sum(rowsum_ref[...], axis=0) + 1e-10
    dinv_ref[...] = lax.rsqrt(s)


def _edge2_body(dinv_hbm, ei_hbm, masked_hbm, out_hbm,
                dinv_v, ei_v, masked_v, out_v, sem):
    cid = lax.axis_index("c")
    sid = lax.axis_index("s")
    wid = sid * _NC + cid
    own_lo = (wid * _NCH) // _NW * _D
    rb = jnp.minimum(own_lo // _D, _NCH - _WIN)
    ebase = pl.multiple_of(rb * _D, _D)
    cps = [pltpu.async_copy(dinv_hbm, dinv_v, sem),
           pltpu.async_copy(ei_hbm.at[pl.ds(rb, _WIN)], ei_v, sem),
           pltpu.async_copy(masked_hbm.at[pl.ds(ebase, _WE)], masked_v, sem)]
    for cp in cps:
        cp.wait()

    @plsc.parallel_loop(0, _WE // _L, unroll=8)
    def body(i):
        s = pl.ds(i * _L, _L)
        c8 = i >> 3
        sl = pl.ds((i & 7) * _L, _L)
        dr = plsc.load_gather(dinv_v, [ei_v[c8, 0, sl]])
        dc = plsc.load_gather(dinv_v, [ei_v[c8, 1, sl]])
        out_v[s] = masked_v[s] * dr * dc

    pltpu.sync_copy(out_v, out_hbm.at[pl.ds(ebase, _WE)])


_mesh = plsc.VectorSubcoreMesh(core_axis_name="c", subcore_axis_name="s",
                               num_cores=_NC, num_subcores=_NS)

_sc_params = pltpu.CompilerParams(needs_layout_passes=False)

_edge1 = pl.kernel(
    _edge1_body,
    out_type=[jax.ShapeDtypeStruct((_E,), jnp.float32),
              jax.ShapeDtypeStruct((_NW, _NP // _D, _D), jnp.float32)],
    mesh=_mesh,
    scratch_types=[pltpu.VMEM((_NP,), jnp.float32),
                   pltpu.VMEM((_NP,), jnp.float32),
                   pltpu.VMEM((_WIN, 2, _D), jnp.int32),
                   pltpu.VMEM((_WIN, 1, _D), jnp.float32),
                   pltpu.VMEM((_WE,), jnp.float32),
                   pltpu.VMEM((_WE,), jnp.float32),
                   pltpu.VMEM((_NP // _D, _D), jnp.float32),
                   pltpu.SemaphoreType.DMA],
    compiler_params=_sc_params,
)

_edge2 = pl.kernel(
    _edge2_body,
    out_type=jax.ShapeDtypeStruct((_E,), jnp.float32),
    mesh=_mesh,
    scratch_types=[pltpu.VMEM((_NP,), jnp.float32),
                   pltpu.VMEM((_WIN, 2, _D), jnp.int32),
                   pltpu.VMEM((_WE,), jnp.float32),
                   pltpu.VMEM((_WE,), jnp.float32),
                   pltpu.SemaphoreType.DMA],
    compiler_params=_sc_params,
)


def kernel(x, edge_index, adj_values, noise, nb_W, nb_b, self_W, self_b,
           att_W, att_b):
    nblk = _NP // _D

    W2 = jnp.concatenate([nb_W, self_W], axis=1)
    b2 = jnp.concatenate([nb_b, self_b]).reshape(2 * _H, 1)
    e1_2d, e2_2d = pl.pallas_call(
        _prep_body,
        grid=(nblk // 40,),
        in_specs=[
            pl.BlockSpec((40 * _D, _D), lambda g: (g, 0)),
            pl.BlockSpec((_D, 2 * _H), lambda g: (0, 0)),
            pl.BlockSpec((2 * _H, 1), lambda g: (0, 0)),
            pl.BlockSpec((2 * _H, 1), lambda g: (0, 0)),
            pl.BlockSpec((1, 1), lambda g: (0, 0)),
        ],
        out_specs=[pl.BlockSpec((40, _D), lambda g: (g, 0)),
                   pl.BlockSpec((40, _D), lambda g: (g, 0))],
        out_shape=[jax.ShapeDtypeStruct((nblk, _D), jnp.float32),
                   jax.ShapeDtypeStruct((nblk, _D), jnp.float32)],
    )(x, W2, b2, att_W, att_b.reshape(1, 1))

    e1 = e1_2d.reshape(_NP)
    e2 = e2_2d.reshape(_NP)

    et = edge_index.reshape(2, _NCH, _D).transpose(1, 0, 2)
    noise3 = noise.reshape(_NCH, 1, _D)
    masked, rowsum_p = _edge1(e1, e2, et, noise3, adj_values)

    dinv2d = pl.pallas_call(
        _norm_body,
        out_shape=jax.ShapeDtypeStruct((nblk, _D), jnp.float32),
    )(rowsum_p)
    dinv = dinv2d.reshape(_NP)

    return _edge2(dinv, et, masked)
